# baseline (speedup 1.0000x reference)
import functools

import jax
import jax.numpy as jnp
from jax import lax
from jax.experimental import pallas as pl
from jax.experimental.pallas import tpu as pltpu
from jax.experimental.pallas import tpu_sc as plsc

N_NODES = 10000
N_EDGES = 320000
D_IN = 128
D_EDGE = 16
D_NODE = 18
H_FEA = 16
G_POOL = 64

DP = 32
HW = DP // 2
CNT = D_NODE

NC = 2
NS = 16
NW = NC * NS
EP = 327680
E8P = EP // 8
RPW = E8P // NW
CH8 = 40
CH = 8 * CH8
NCHUNK = RPW // CH8
NP = 10240
RPS = NP // NS
DISCARD = N_NODES


def _pack_pair(lo, hi):
    ulo = lax.bitcast_convert_type(lo, jnp.uint32)
    uhi = lax.bitcast_convert_type(hi, jnp.uint32)
    rlo = (ulo + 0x7FFF + ((ulo >> 16) & 1)) >> 16
    rhi = (uhi + 0x7FFF + ((uhi >> 16) & 1)) >> 16
    return lax.bitcast_convert_type((rhi << 16) | rlo, jnp.float32)


def _node_tables_body(x_ref, w_ref, a_ref, b_ref):
    ab = jnp.dot(x_ref[...], w_ref[...], preferred_element_type=jnp.float32)
    a_ref[...] = _pack_pair(ab[:, 0:HW], ab[:, HW:DP])
    b_ref[...] = _pack_pair(ab[:, DP:DP + HW], ab[:, DP + HW:])


def _node_tables(x, wab):
    return pl.pallas_call(
        _node_tables_body,
        out_shape=(
            jax.ShapeDtypeStruct((NP, HW), jnp.float32),
            jax.ShapeDtypeStruct((NP, HW), jnp.float32),
        ),
    )(x, wab)


_RBLK = 2560


def _edge_table_body(e0, e1, e2, e3, e4, e5, e6, e7, w_ref, bias_ref, c_ref):
    los, his = [], []
    for ea_ref in (e0, e1, e2, e3, e4, e5, e6, e7):
        cm = (jnp.dot(ea_ref[...], w_ref[...],
                      preferred_element_type=jnp.float32) + bias_ref[...])
        los.append(cm[:, 0:HW])
        his.append(cm[:, HW:DP])
    c_ref[...] = _pack_pair(jnp.concatenate(los, axis=1),
                            jnp.concatenate(his, axis=1))


def _edge_table(edge_attr, w3p, cbias):
    grid = E8P // _RBLK
    nblk = N_EDGES // _RBLK
    oct_spec = lambda o: pl.BlockSpec(
        (_RBLK, D_EDGE),
        lambda i, o=o: (jnp.minimum(o * grid + i, nblk - 1), 0))
    return pl.pallas_call(
        _edge_table_body,
        grid=(grid,),
        in_specs=[oct_spec(o) for o in range(8)] + [
            pl.BlockSpec((D_EDGE, DP), lambda i: (0, 0)),
            pl.BlockSpec((1, DP), lambda i: (0, 0)),
        ],
        out_specs=pl.BlockSpec((_RBLK, 8 * HW), lambda i: (i, 0)),
        out_shape=jax.ShapeDtypeStruct((E8P, 8 * HW), jnp.float32),
    )(*([edge_attr] * 8), w3p, cbias)


def _edge_agg_body(idx_hbm, a_hbm, b_hbm, c_hbm, out_hbm,
                   dst_all, src_all, a0, a1, b0, b1, c0, c1, m0, m1,
                   z_v, acc_sh, sa0, sa1, sb0, sb1, sc0, sc1, ss0, ss1):
    cid = lax.axis_index("c")
    sid = lax.axis_index("s")
    wid = cid * NS + sid
    slots = ((a0, b0, c0, m0, sa0, sb0, sc0, ss0),
             (a1, b1, c1, m1, sa1, sb1, sc1, ss1))

    pltpu.sync_copy(idx_hbm.at[wid], src_all)
    pltpu.sync_copy(idx_hbm.at[NW + wid], dst_all)

    def gathers(cn, slot):
        av, bv, cv = slot[0], slot[1], slot[2]
        sa, sb, sc = slot[4], slot[5], slot[6]
        pltpu.async_copy(a_hbm.at[dst_all.at[cn]], av, sa)
        pltpu.async_copy(b_hbm.at[src_all.at[cn]], bv, sb)
        pltpu.async_copy(
            c_hbm.at[pl.ds(wid * (RPW * 128) + cn * (CH8 * 128), CH8 * 128)],
            cv, sc)

    def wait_gathers(cn, slot):
        av, bv, cv = slot[0], slot[1], slot[2]
        sa, sb, sc = slot[4], slot[5], slot[6]
        pltpu.make_async_copy(a_hbm.at[dst_all.at[cn]], av, sa).wait()
        pltpu.make_async_copy(b_hbm.at[src_all.at[cn]], bv, sb).wait()
        pltpu.make_async_copy(
            c_hbm.at[pl.ds(wid * (RPW * 128) + cn * (CH8 * 128), CH8 * 128)],
            cv, sc).wait()

    def wait_scatter(cp, slot):
        mv, ss = slot[3], slot[7]
        pltpu.make_async_copy(mv, acc_sh.at[dst_all.at[cp]], ss).wait()

    gathers(0, slots[0])

    def zrow(j, carry):
        z_v[j, pl.ds(0, 16)] = jnp.zeros((16,), jnp.float32)
        z_v[j, pl.ds(16, 16)] = jnp.zeros((16,), jnp.float32)
        return carry

    lax.fori_loop(0, RPS, zrow, 0)
    pltpu.sync_copy(z_v, acc_sh.at[pl.ds(sid * RPS, RPS)])
    plsc.subcore_barrier()

    def pair(t, carry):
        for b in (0, 1):
            ci = 2 * t + b
            nxt = ci + 1
            cur, nsl = slots[b], slots[1 - b]

            @pl.when(nxt < NCHUNK)
            def _prefetch():
                @pl.when(ci >= 1)
                def _free():
                    wait_scatter(ci - 1, nsl)
                gathers(nxt, nsl)

            wait_gathers(ci, cur)
            av, bv, cv, mv = cur[0], cur[1], cur[2], cur[3]

            def mrow(r, inner):
                for o in range(8):
                    j = o * CH8 + r
                    ap = plsc.bitcast(av[j, pl.ds(0, HW)], jnp.bfloat16)
                    bp = plsc.bitcast(bv[j, pl.ds(0, HW)], jnp.bfloat16)
                    cp = plsc.bitcast(cv[pl.ds(r * 128 + o * HW, HW)],
                                      jnp.bfloat16)
                    alo, ahi = plsc.unpack(ap,
                                           format=plsc.PackFormat.INTERLEAVED)
                    blo, bhi = plsc.unpack(bp,
                                           format=plsc.PackFormat.INTERLEAVED)
                    clo, chi = plsc.unpack(cp,
                                           format=plsc.PackFormat.INTERLEAVED)
                    mv[j, pl.ds(0, 16)] = jnp.maximum(alo + blo + clo, 0.0)
                    mv[j, pl.ds(16, 16)] = jnp.maximum(ahi + bhi + chi, 0.0)
                return inner

            lax.fori_loop(0, CH8, mrow, 0)
            pltpu.async_copy(mv, acc_sh.at[dst_all.at[ci]], cur[7], add=True)
        return carry

    lax.fori_loop(0, NCHUNK // 2, pair, 0)
    wait_scatter(0, slots[0])
    wait_scatter(0, slots[1])
    plsc.subcore_barrier()

    pltpu.sync_copy(acc_sh.at[pl.ds(sid * RPS, RPS)], z_v)
    pltpu.sync_copy(z_v, out_hbm.at[cid, pl.ds(sid * RPS, RPS)])


def _edge_agg(idx3, a, b, c_flat):
    mesh = plsc.VectorSubcoreMesh(
        core_axis_name="c", subcore_axis_name="s",
        num_cores=NC, num_subcores=NS,
    )
    f = functools.partial(
        pl.kernel,
        out_type=jax.ShapeDtypeStruct((NC, NP, DP), jnp.float32),
        mesh=mesh,
        scratch_types=[
            pltpu.VMEM((NCHUNK, CH), jnp.int32),
            pltpu.VMEM((NCHUNK, CH), jnp.int32),
            pltpu.VMEM((CH, HW), jnp.float32),
            pltpu.VMEM((CH, HW), jnp.float32),
            pltpu.VMEM((CH, HW), jnp.float32),
            pltpu.VMEM((CH, HW), jnp.float32),
            pltpu.VMEM((CH8 * 128,), jnp.float32),
            pltpu.VMEM((CH8 * 128,), jnp.float32),
            pltpu.VMEM((CH, DP), jnp.float32),
            pltpu.VMEM((CH, DP), jnp.float32),
            pltpu.VMEM((RPS, DP), jnp.float32),
            pltpu.VMEM_SHARED((NP, DP), jnp.float32),
            pltpu.SemaphoreType.DMA,
            pltpu.SemaphoreType.DMA,
            pltpu.SemaphoreType.DMA,
            pltpu.SemaphoreType.DMA,
            pltpu.SemaphoreType.DMA,
            pltpu.SemaphoreType.DMA,
            pltpu.SemaphoreType.DMA,
            pltpu.SemaphoreType.DMA,
        ],
        compiler_params=pltpu.CompilerParams(use_tc_tiling_on_sc=False,
                                             needs_layout_passes=False),
    )(_edge_agg_body)
    return f(idx3, a, b, c_flat)


def _tail_body(p_ref, batch_ref, wpT_ref, bp_ref, w1T_ref, b1_ref,
               w2T_ref, b2_ref, w3T_ref, b3_ref, out_ref):
    s = p_ref[0, :N_NODES] + p_ref[1, :N_NODES]
    cnt = jnp.maximum(s[:, CNT], 1.0)
    h2 = s[:, :D_NODE] / cnt[:, None]
    hp = jnp.dot(h2, wpT_ref[...], preferred_element_type=jnp.float32) + bp_ref[...]
    h3 = jnp.maximum(hp, 0.0) + jnp.log1p(jnp.exp(-jnp.abs(hp)))
    gids = lax.broadcasted_iota(jnp.int32, (N_NODES, G_POOL), 1)
    onehot = (batch_ref[...][:, None] == gids).astype(jnp.float32)
    psum = lax.dot_general(onehot, h3, (((0,), (0,)), ((), ())),
                           preferred_element_type=jnp.float32)
    pcnt = jnp.maximum(jnp.sum(onehot, axis=0), 1.0)
    pooled = psum / pcnt[:, None]
    o = jnp.maximum(jnp.dot(pooled, w1T_ref[...]) + b1_ref[...], 0.0)
    o = jnp.maximum(jnp.dot(o, w2T_ref[...]) + b2_ref[...], 0.0)
    out_ref[...] = jnp.dot(o, w3T_ref[...]) + b3_ref[...]


def _tail(parts, batch, wpT, bp, w1T, b1, w2T, b2, w3T, b3):
    return pl.pallas_call(
        _tail_body,
        out_shape=jax.ShapeDtypeStruct((G_POOL, 1), jnp.float32),
    )(parts, batch, wpT, bp, w1T, b1, w2T, b2, w3T, b3)


def kernel(x, edge_index, edge_attr, batch, W_emb, b_emb, W_conv, b_conv,
           W_post, b_post, W_f1, b_f1, W_f2, b_f2, W_f3, b_f3):
    W1 = W_conv[:, :D_NODE]
    W2 = W_conv[:, D_NODE:2 * D_NODE]
    W3 = W_conv[:, 2 * D_NODE:]
    WA = W1 @ W_emb
    WB = W2 @ W_emb
    bA = W1 @ b_emb
    bB = W2 @ b_emb
    wab = jnp.zeros((D_IN, 2 * DP), jnp.float32)
    wab = wab.at[:, :D_NODE].set(WA.T)
    wab = wab.at[:, DP:DP + D_NODE].set(WB.T)
    w3p = jnp.zeros((D_EDGE, DP), jnp.float32)
    w3p = w3p.at[:, :D_NODE].set(W3.T)
    cbias = jnp.zeros((DP,), jnp.float32)
    cbias = cbias.at[:D_NODE].set(b_conv + bA + bB)
    cbias = cbias.at[CNT].set(1.0)

    xp = jnp.pad(x, ((0, NP - N_NODES), (0, 0)))
    idxp = jnp.pad(edge_index, ((0, 0), (0, EP - N_EDGES)))
    idxp = idxp.at[1, N_EDGES:].set(DISCARD)
    idxp = idxp.reshape(2, 8, NW, NCHUNK, CH8).transpose(0, 2, 3, 1, 4)
    idx3 = idxp.reshape(2 * NW, NCHUNK, CH)

    a_tab, b_tab = _node_tables(xp, wab)
    c_tab = _edge_table(edge_attr, w3p, cbias[None, :])
    parts = _edge_agg(idx3, a_tab, b_tab, c_tab.reshape(-1))
    return _tail(parts, batch, W_post.T, b_post, W_f1.T, b_f1,
                 W_f2.T, b_f2, W_f3.T, b_f3)


# speedup vs baseline: 1.0934x; 1.0934x over previous
import functools

import jax
import jax.numpy as jnp
from jax import lax
from jax.experimental import pallas as pl
from jax.experimental.pallas import tpu as pltpu
from jax.experimental.pallas import tpu_sc as plsc

N_NODES = 10000
N_EDGES = 320000
D_IN = 128
D_EDGE = 16
D_NODE = 18
H_FEA = 16
G_POOL = 64

DP = 32
CNT = D_NODE

NC = 2
NS = 16
NW = NC * NS
EPW = 10240
CH = 320
NCHUNK = EPW // CH
CH4 = CH // 4
NP = 10240
RPS = NP // NS


def _node_tables_body(x_ref, w_ref, a_ref, b_ref):
    ab = jnp.dot(x_ref[...], w_ref[...], preferred_element_type=jnp.float32)
    a_ref[...] = ab[:, :DP]
    b_ref[...] = ab[:, DP:]


def _node_tables(x, wab):
    return pl.pallas_call(
        _node_tables_body,
        out_shape=(
            jax.ShapeDtypeStruct((NP, DP), jnp.float32),
            jax.ShapeDtypeStruct((NP, DP), jnp.float32),
        ),
    )(x, wab)


E4 = N_EDGES // 4
_EBLK4 = 5000
_NBLK = E4 // _EBLK4


def _edge_table_body(e0, e1, e2, e3, w_ref, bias_ref, c_ref):
    parts = []
    for ea_ref in (e0, e1, e2, e3):
        parts.append(
            jnp.dot(ea_ref[...], w_ref[...], preferred_element_type=jnp.float32)
            + bias_ref[...]
        )
    c_ref[...] = jnp.concatenate(parts, axis=1)


def _edge_table(edge_attr, w3p, cbias):
    quarter_spec = lambda q: pl.BlockSpec(
        (_EBLK4, D_EDGE), lambda i, q=q: (q * _NBLK + i, 0))
    return pl.pallas_call(
        _edge_table_body,
        grid=(_NBLK,),
        in_specs=[
            quarter_spec(0), quarter_spec(1), quarter_spec(2), quarter_spec(3),
            pl.BlockSpec((D_EDGE, DP), lambda i: (0, 0)),
            pl.BlockSpec((1, DP), lambda i: (0, 0)),
        ],
        out_specs=pl.BlockSpec((_EBLK4, 4 * DP), lambda i: (i, 0)),
        out_shape=jax.ShapeDtypeStruct((E4, 4 * DP), jnp.float32),
    )(edge_attr, edge_attr, edge_attr, edge_attr, w3p, cbias)


def _edge_agg_body(idx_hbm, a_hbm, b_hbm, c_hbm, out_hbm,
                   dst_all, src_all, a0, a1, b0, b1, c0, c1, z_v, acc_sh,
                   sa0, sa1, sb0, sb1, sc0, sc1, ss0, ss1):
    cid = lax.axis_index("c")
    sid = lax.axis_index("s")
    wid = cid * NS + sid
    slots = ((a0, b0, c0, sa0, sb0, sc0, ss0),
             (a1, b1, c1, sa1, sb1, sc1, ss1))
    nchunk = jnp.where(wid == NW - 1, (N_EDGES - (NW - 1) * EPW) // CH, NCHUNK)

    pltpu.sync_copy(idx_hbm.at[wid], src_all)
    pltpu.sync_copy(idx_hbm.at[NW + wid], dst_all)

    def gathers(cn, slot):
        av, bv, cv, sa, sb, sc, _ = slot
        pltpu.async_copy(a_hbm.at[dst_all.at[cn]], av, sa)
        pltpu.async_copy(b_hbm.at[src_all.at[cn]], bv, sb)
        pltpu.async_copy(
            c_hbm.at[pl.ds(wid * (EPW * DP) + cn * (CH * DP), CH * DP)], cv, sc)

    def wait_gathers(cn, slot):
        av, bv, cv, sa, sb, sc, _ = slot
        pltpu.make_async_copy(a_hbm.at[dst_all.at[cn]], av, sa).wait()
        pltpu.make_async_copy(b_hbm.at[src_all.at[cn]], bv, sb).wait()
        pltpu.make_async_copy(
            c_hbm.at[pl.ds(wid * (EPW * DP) + cn * (CH * DP), CH * DP)],
            cv, sc).wait()

    def wait_scatter(cp, slot):
        av, ss = slot[0], slot[6]
        pltpu.make_async_copy(av, acc_sh.at[dst_all.at[cp]], ss).wait()

    gathers(0, slots[0])

    def zrow(j, carry):
        z_v[j, pl.ds(0, 16)] = jnp.zeros((16,), jnp.float32)
        z_v[j, pl.ds(16, 16)] = jnp.zeros((16,), jnp.float32)
        return carry

    lax.fori_loop(0, RPS, zrow, 0)
    pltpu.sync_copy(z_v, acc_sh.at[pl.ds(sid * RPS, RPS)])
    plsc.subcore_barrier()

    def pair(t, carry):
        for b in (0, 1):
            ci = 2 * t + b
            nxt = ci + 1
            cur, nsl = slots[b], slots[1 - b]

            @pl.when(nxt < nchunk)
            def _prefetch():
                @pl.when(ci >= 1)
                def _free():
                    wait_scatter(ci - 1, nsl)
                gathers(nxt, nsl)

            wait_gathers(ci, cur)
            av, bv, cv = cur[0], cur[1], cur[2]

            def mrow(r, inner):
                for q in range(4):
                    for k in range(DP // 16):
                        sl = pl.ds(k * 16, 16)
                        j = q * CH4 + r
                        m = (av[j, sl] + bv[j, sl]
                             + cv[pl.ds(r * 128 + q * DP + k * 16, 16)])
                        av[j, sl] = jnp.maximum(m, 0.0)
                return inner

            lax.fori_loop(0, CH4, mrow, 0)
            pltpu.async_copy(av, acc_sh.at[dst_all.at[ci]], cur[6], add=True)
        return carry

    lax.fori_loop(0, nchunk // 2, pair, 0)
    wait_scatter(0, slots[0])
    wait_scatter(0, slots[1])
    plsc.subcore_barrier()

    pltpu.sync_copy(acc_sh.at[pl.ds(sid * RPS, RPS)], z_v)
    pltpu.sync_copy(z_v, out_hbm.at[cid, pl.ds(sid * RPS, RPS)])


def _edge_agg(idx3, a, b, c_flat):
    mesh = plsc.VectorSubcoreMesh(
        core_axis_name="c", subcore_axis_name="s",
        num_cores=NC, num_subcores=NS,
    )
    f = functools.partial(
        pl.kernel,
        out_type=jax.ShapeDtypeStruct((NC, NP, DP), jnp.float32),
        mesh=mesh,
        scratch_types=[
            pltpu.VMEM((NCHUNK, CH), jnp.int32),
            pltpu.VMEM((NCHUNK, CH), jnp.int32),
            pltpu.VMEM((CH, DP), jnp.float32),
            pltpu.VMEM((CH, DP), jnp.float32),
            pltpu.VMEM((CH, DP), jnp.float32),
            pltpu.VMEM((CH, DP), jnp.float32),
            pltpu.VMEM((CH * DP,), jnp.float32),
            pltpu.VMEM((CH * DP,), jnp.float32),
            pltpu.VMEM((RPS, DP), jnp.float32),
            pltpu.VMEM_SHARED((NP, DP), jnp.float32),
            pltpu.SemaphoreType.DMA,
            pltpu.SemaphoreType.DMA,
            pltpu.SemaphoreType.DMA,
            pltpu.SemaphoreType.DMA,
            pltpu.SemaphoreType.DMA,
            pltpu.SemaphoreType.DMA,
            pltpu.SemaphoreType.DMA,
            pltpu.SemaphoreType.DMA,
        ],
        compiler_params=pltpu.CompilerParams(use_tc_tiling_on_sc=False),
    )(_edge_agg_body)
    return f(idx3, a, b, c_flat)


def _tail_body(p_ref, batch_ref, wpT_ref, bp_ref, w1T_ref, b1_ref,
               w2T_ref, b2_ref, w3T_ref, b3_ref, out_ref):
    s = p_ref[0, :N_NODES] + p_ref[1, :N_NODES]
    cnt = jnp.maximum(s[:, CNT], 1.0)
    h2 = s[:, :D_NODE] / cnt[:, None]
    hp = jnp.dot(h2, wpT_ref[...], preferred_element_type=jnp.float32) + bp_ref[...]
    h3 = jnp.maximum(hp, 0.0) + jnp.log1p(jnp.exp(-jnp.abs(hp)))
    gids = lax.broadcasted_iota(jnp.int32, (N_NODES, G_POOL), 1)
    onehot = (batch_ref[...][:, None] == gids).astype(jnp.float32)
    psum = lax.dot_general(onehot, h3, (((0,), (0,)), ((), ())),
                           preferred_element_type=jnp.float32)
    pcnt = jnp.maximum(jnp.sum(onehot, axis=0), 1.0)
    pooled = psum / pcnt[:, None]
    o = jnp.maximum(jnp.dot(pooled, w1T_ref[...]) + b1_ref[...], 0.0)
    o = jnp.maximum(jnp.dot(o, w2T_ref[...]) + b2_ref[...], 0.0)
    out_ref[...] = jnp.dot(o, w3T_ref[...]) + b3_ref[...]


def _tail(parts, batch, wpT, bp, w1T, b1, w2T, b2, w3T, b3):
    return pl.pallas_call(
        _tail_body,
        out_shape=jax.ShapeDtypeStruct((G_POOL, 1), jnp.float32),
    )(parts, batch, wpT, bp, w1T, b1, w2T, b2, w3T, b3)


def kernel(x, edge_index, edge_attr, batch, W_emb, b_emb, W_conv, b_conv,
           W_post, b_post, W_f1, b_f1, W_f2, b_f2, W_f3, b_f3):
    W1 = W_conv[:, :D_NODE]
    W2 = W_conv[:, D_NODE:2 * D_NODE]
    W3 = W_conv[:, 2 * D_NODE:]
    WA = W1 @ W_emb
    WB = W2 @ W_emb
    bA = W1 @ b_emb
    bB = W2 @ b_emb
    wab = jnp.zeros((D_IN, 2 * DP), jnp.float32)
    wab = wab.at[:, :D_NODE].set(WA.T)
    wab = wab.at[:, DP:DP + D_NODE].set(WB.T)
    w3p = jnp.zeros((D_EDGE, DP), jnp.float32)
    w3p = w3p.at[:, :D_NODE].set(W3.T)
    cbias = jnp.zeros((DP,), jnp.float32)
    cbias = cbias.at[:D_NODE].set(b_conv + bA + bB)
    cbias = cbias.at[CNT].set(1.0)

    xp = jnp.pad(x, ((0, NP - N_NODES), (0, 0)))
    idxp = edge_index.reshape(2, 4, E4)
    idxp = jnp.pad(idxp, ((0, 0), (0, 0), (0, NW * EPW // 4 - E4)))
    idxp = idxp.reshape(2, 4, NW, NCHUNK, CH4).transpose(0, 2, 3, 1, 4)
    idx3 = idxp.reshape(2 * NW, NCHUNK, CH)

    a_tab, b_tab = _node_tables(xp, wab)
    c_tab = _edge_table(edge_attr, w3p, cbias[None, :])
    parts = _edge_agg(idx3, a_tab, b_tab, c_tab.reshape(-1))
    return _tail(parts, batch, W_post.T, b_post, W_f1.T, b_f1,
                 W_f2.T, b_f2, W_f3.T, b_f3)
